# Initial kernel scaffold; baseline (speedup 1.0000x reference)
import jax
import jax.numpy as jnp
from jax.experimental import pallas as pl


def kernel(x, edge_index, i, params):
    raise NotImplementedError("write your pallas kernel here")


# speedup vs baseline: 6.9524x; 6.9524x over previous
import functools
import math

import jax
import jax.numpy as jnp
from jax import lax
from jax.experimental import pallas as pl
from jax.experimental.pallas import tpu as pltpu
from jax.experimental.pallas import tpu_sc as plsc

EPS = 1e-3
N = 10000
E = 320000
NP = 10240
HALF = NP // 2
NC, NS = 2, 16
ACC_R = HALF + 64
DEG_R = 16 * 656
DEG_INIT = DEG_R // NS
EPT_P = E // (NC * NS)
EPT_M = E // NS

DEG_CH = 80
KCH = 64
CAP = EPT_P + 2 * KCH
BR = NP // 8

def _mesh():
    return plsc.VectorSubcoreMesh(
        core_axis_name="c", subcore_axis_name="s",
        num_cores=NC, num_subcores=NS)


@functools.cache
def _edge_prep_krn():
  return functools.partial(
    pl.kernel,
    out_type=(
        jax.ShapeDtypeStruct((E,), jnp.int32),
        jax.ShapeDtypeStruct((E,), jnp.int32),
        jax.ShapeDtypeStruct((NC, DEG_R), jnp.float32),
    ),
    mesh=_mesh(),
    compiler_params=pltpu.CompilerParams(needs_layout_passes=False),
    scratch_types=[
        pltpu.VMEM((NP,), jnp.float32),
        pltpu.VMEM((EPT_P,), jnp.int32),
        pltpu.VMEM((EPT_P,), jnp.int32),
        pltpu.VMEM((EPT_P,), jnp.int32),
        pltpu.VMEM((EPT_P,), jnp.int32),
        pltpu.VMEM((EPT_P,), jnp.float32),
        pltpu.VMEM((EPT_P,), jnp.int32),
        pltpu.VMEM((DEG_CH,), jnp.int32),
        pltpu.VMEM((DEG_INIT,), jnp.float32),
        pltpu.VMEM((DEG_R,), jnp.float32),
        pltpu.VMEM_SHARED((DEG_R,), jnp.float32),
        pltpu.SemaphoreType.DMA,
    ],
  )(_edge_prep_body)


def _edge_prep(*args):
    return _edge_prep_krn()(*args)


def _edge_prep_body(src_h, dst_h, m_h, srcp_h, dstp_h, deg_h,
               m_v, src_v, dst_v, srcp_v, dstp_v, val_v, didx_v, istg_v,
               zero_v, wb_v, acc_sh, sem):
    c = lax.axis_index("c")
    s = lax.axis_index("s")
    wid = s * NC + c
    lane = lax.iota(jnp.int32, 16)

    for j in range(DEG_INIT // 16):
        zero_v[pl.ds(j * 16, 16)] = jnp.zeros((16,), jnp.float32)
    pltpu.sync_copy(zero_v, acc_sh.at[pl.ds(s * DEG_INIT, DEG_INIT)])

    pltpu.sync_copy(m_h, m_v)
    base = wid * EPT_P
    pltpu.sync_copy(src_h.at[pl.ds(base, EPT_P)], src_v)
    pltpu.sync_copy(dst_h.at[pl.ds(base, EPT_P)], dst_v)
    plsc.subcore_barrier()

    def body(j, carry):
        sl = pl.ds(j * 16, 16)
        sv = src_v[sl]
        dv = dst_v[sl]
        msrc = plsc.load_gather(m_v, [sv])
        mdst = plsc.load_gather(m_v, [dv])
        valid = (msrc > 0.5) & (mdst > 0.5)
        srcp_v[sl] = jnp.where(valid, sv, NP)
        dstp_v[sl] = jnp.where(valid, dv, NP)
        didx_v[sl] = jnp.where(valid, dv, NP + lane)
        val_v[sl] = jnp.where(valid, 1.0, 0.0)
        return carry

    lax.fori_loop(0, EPT_P // 16, body, 0)

    pltpu.sync_copy(srcp_v, srcp_h.at[pl.ds(base, EPT_P)])
    pltpu.sync_copy(dstp_v, dstp_h.at[pl.ds(base, EPT_P)])

    def dbody(g, carry):
        gb = g * DEG_CH
        for t in range(DEG_CH // 16):
            istg_v[pl.ds(t * 16, 16)] = didx_v[pl.ds(gb + t * 16, 16)]
        pltpu.sync_copy(val_v.at[pl.ds(gb, DEG_CH)], acc_sh.at[istg_v],
                        add=True)
        return carry

    lax.fori_loop(0, EPT_P // DEG_CH, dbody, 0)
    plsc.subcore_barrier()

    @pl.when(s == 0)
    def _():
        pltpu.sync_copy(acc_sh, wb_v)
        pltpu.sync_copy(wb_v, deg_h.at[c])


ROWS_T = NP // (NC * NS)
SSEC = 6400
NSEC = E // SSEC
SCAP = SSEC + KCH


@functools.cache
def _msgpass_krn():
  return functools.partial(
    pl.kernel,
    out_type=jax.ShapeDtypeStruct((NP, 256), jnp.float32),
    mesh=_mesh(),
    compiler_params=pltpu.CompilerParams(needs_layout_passes=False),
    scratch_types=[
        pltpu.VMEM((SSEC,), jnp.int32),
        pltpu.VMEM((SSEC,), jnp.int32),
        pltpu.VMEM((SCAP,), jnp.int32),
        pltpu.VMEM((SCAP,), jnp.int32),
        pltpu.VMEM((KCH, 256), jnp.float32),
        pltpu.VMEM((KCH,), jnp.int32),
        pltpu.VMEM((ROWS_T + 16, 256), jnp.float32),
        pltpu.SemaphoreType.DMA,
    ],
  )(_msgpass_body)


def _msgpass(*args):
    return _msgpass_krn()(*args)


def _msgpass_body(yh_h, srcp_h, dstp_h, msum_h,
                  src_v, dst_v, csrc_v, cdst_v, rows_v, isrc_v, acc_v, sem):
    c = lax.axis_index("c")
    s = lax.axis_index("s")
    wid = s * NC + c
    lane = lax.iota(jnp.int32, 16)
    lo = wid * ROWS_T
    hi = jnp.minimum(lo + ROWS_T, N)

    z16 = jnp.zeros((16,), jnp.float32)

    def zbody(r, carry):
        for t in range(16):
            acc_v[r, pl.ds(t * 16, 16)] = z16
        return carry

    lax.fori_loop(0, ROWS_T + 16, zbody, 0)

    def section(sec, carry):
        base = sec * SSEC
        pltpu.sync_copy(srcp_h.at[pl.ds(base, SSEC)], src_v)
        pltpu.sync_copy(dstp_h.at[pl.ds(base, SSEC)], dst_v)

        def cbody(j, off):
            sl = pl.ds(j * 16, 16)
            dv = dst_v[sl]
            owned = (dv >= lo) & (dv < hi)
            sv = src_v[sl]
            plsc.store_compressed(csrc_v.at[pl.ds(off, 16)], sv, mask=owned)
            plsc.store_compressed(cdst_v.at[pl.ds(off, 16)], dv - lo,
                                  mask=owned)
            cnt = plsc.all_reduce_population_count(owned)
            return off + jnp.max(cnt)

        cnt = lax.fori_loop(0, SSEC // 16, cbody, jnp.int32(0))

        for j in range(KCH // 16):
            csrc_v[pl.ds(cnt + j * 16, 16)] = (NP - 16) + lane
            cdst_v[pl.ds(cnt + j * 16, 16)] = ROWS_T + lane

        nch = (cnt + (KCH - 1)) // KCH

        def gbody(g, carry2):
            gb = g * KCH
            for t in range(KCH // 16):
                isrc_v[pl.ds(t * 16, 16)] = csrc_v[pl.ds(gb + t * 16, 16)]
            pltpu.async_copy(yh_h.at[isrc_v], rows_v, sem).wait()

            def abody(q, carry3):
                dlv = cdst_v[pl.ds(gb + q * 16, 16)]
                for r16 in range(16):
                    dl = dlv[r16]
                    r = q * 16 + r16
                    for t in range(16):
                        sl = pl.ds(t * 16, 16)
                        plsc.addupdate(acc_v.at[dl, sl], rows_v[r, sl])
                return carry3

            lax.fori_loop(0, KCH // 16, abody, 0)
            return carry2

        lax.fori_loop(0, nch, gbody, 0)
        return carry

    lax.fori_loop(0, NSEC, section, 0)

    pltpu.sync_copy(acc_v.at[pl.ds(0, ROWS_T)],
                    msum_h.at[pl.ds(lo, ROWS_T)])


_SB = 1.0 / math.sqrt(1.0 + EPS)


def _tc_call(body, grid, in_specs, out_specs, out_shape):
    return pl.pallas_call(body, grid=grid, in_specs=in_specs,
                          out_specs=out_specs, out_shape=out_shape)


def _full(shape):
    return pl.BlockSpec(shape, lambda i: tuple(0 for _ in shape))


def _rows(shape):
    return pl.BlockSpec(shape, lambda i: (i,) + tuple(0 for _ in shape[1:]))


def _rows_hi(shape):
    return pl.BlockSpec(shape, lambda i: (i + 8,) + tuple(0 for _ in shape[1:]))


def _pre_mlp_body(x_ref, a0, c0, a1, c1, a2, c2, out_ref):
    h = jnp.tanh(x_ref[...] @ a0[...] + c0[...])
    h = jnp.tanh(h @ a1[...] + c1[...])
    out_ref[...] = jnp.tanh(h @ a2[...] + c2[...])


def _pre_mlp(x, a0, c0, a1, c1, a2, c2):
    return _tc_call(
        _pre_mlp_body, (8,),
        [_rows((BR, 128)), _full((128, 256)), _full((1, 256)),
         _full((256, 256)), _full((1, 256)), _full((256, 256)),
         _full((1, 256))],
        _rows((BR, 256)), jax.ShapeDtypeStruct((NP, 256), jnp.float32),
    )(x, a0, c0, a1, c1, a2, c2)


def _deg_reduce_body(dp_ref, dinv_ref, dgi_ref):
    d = 1.0 + dp_ref[0] + dp_ref[1]
    dinv_ref[...] = lax.rsqrt(d)
    dgi_ref[...] = 1.0 / d


def _deg_reduce(dp):
    return _tc_call(
        _deg_reduce_body, (1,),
        [_full((NC, 80, 128))],
        (_full((80, 128)), _full((80, 128))),
        (jax.ShapeDtypeStruct((80, 128), jnp.float32),
         jax.ShapeDtypeStruct((80, 128), jnp.float32)),
    )(dp)


def _rowmask(i):
    r = i * BR + lax.broadcasted_iota(jnp.int32, (BR, 1), 0)
    return (r < N).astype(jnp.float32)


def _conv_pre_body(x_ref, w_ref, dinv_ref, y_ref, yh_ref):
    i = pl.program_id(0)
    y = x_ref[...] @ w_ref[...]
    y_ref[...] = y
    yh_ref[...] = y * dinv_ref[...] * _rowmask(i)


def _conv_pre(x, w, dinv):
    return _tc_call(
        _conv_pre_body, (8,),
        [_rows((BR, 256)), _full((256, 256)), _rows((BR, 1))],
        (_rows((BR, 256)), _rows((BR, 256))),
        (jax.ShapeDtypeStruct((NP, 256), jnp.float32),
         jax.ShapeDtypeStruct((NP, 256), jnp.float32)),
    )(x, w, dinv)


def _combine_z(ms_ref, y_ref, dinv_ref, dgi_ref, b_ref, m_ref):
    z = jnp.tanh(dinv_ref[...] * ms_ref[...] + y_ref[...] * dgi_ref[...]
                 + b_ref[...])
    return z * m_ref[...]


def _combine_pre_body(ms_ref, y_ref, dinv_ref, dgi_ref, b_ref,
                      m_ref, w_ref, y2_ref, yh2_ref):
    i = pl.program_id(0)
    z = _combine_z(ms_ref, y_ref, dinv_ref, dgi_ref, b_ref, m_ref)
    y2 = z @ w_ref[...]
    y2_ref[...] = y2
    yh2_ref[...] = y2 * dinv_ref[...] * _rowmask(i)


def _combine_pre(msum, y, dinv, dgi, b, m, w):
    return _tc_call(
        _combine_pre_body, (8,),
        [_rows((BR, 256)), _rows((BR, 256)),
         _rows((BR, 1)), _rows((BR, 1)),
         _full((1, 256)), _rows((BR, 1)), _full((256, 256))],
        (_rows((BR, 256)), _rows((BR, 256))),
        (jax.ShapeDtypeStruct((NP, 256), jnp.float32),
         jax.ShapeDtypeStruct((NP, 256), jnp.float32)),
    )(msum, y, dinv, dgi, b, m, w)


def _combine_score_body(ms_ref, y_ref, dinv_ref, dgi_ref, b_ref,
                        m_ref, p_ref, z_ref, sc_ref):
    z = _combine_z(ms_ref, y_ref, dinv_ref, dgi_ref, b_ref, m_ref)
    z_ref[...] = z
    sc_ref[...] = z @ p_ref[...]


def _combine_score(msum, y, dinv, dgi, b, m, p):
    return _tc_call(
        _combine_score_body, (8,),
        [_rows((BR, 256)), _rows((BR, 256)),
         _rows((BR, 1)), _rows((BR, 1)),
         _full((1, 256)), _rows((BR, 1)), _full((256, 1))],
        (_rows((BR, 256)), _rows((BR, 1))),
        (jax.ShapeDtypeStruct((NP, 256), jnp.float32),
         jax.ShapeDtypeStruct((NP, 1), jnp.float32)),
    )(msum, y, dinv, dgi, b, m, p)


def _combine_read_body(ms_ref, y_ref, dinv_ref, dgi_ref, b_ref,
                       m_ref, kinv_ref, r_ref):
    i = pl.program_id(0)
    z = _combine_z(ms_ref, y_ref, dinv_ref, dgi_ref, b_ref, m_ref)
    ps = jnp.sum(z, axis=0, keepdims=True)
    part = jnp.concatenate([ps * kinv_ref[0, 0], ps], axis=-1)

    @pl.when(i == 0)
    def _():
        r_ref[...] = jnp.zeros_like(r_ref)

    r_ref[...] += part


def _combine_read(msum, y, dinv, dgi, b, m, kinv):
    return _tc_call(
        _combine_read_body, (8,),
        [_rows((BR, 256)), _rows((BR, 256)),
         _rows((BR, 1)), _rows((BR, 1)),
         _full((1, 256)), _rows((BR, 1)), _full((1, 1))],
        _full((1, 512)), jax.ShapeDtypeStruct((1, 512), jnp.float32),
    )(msum, y, dinv, dgi, b, m, kinv)


def _pool_body(sc_ref, mprev_ref, k_ref, mnew_ref):
    k = k_ref[0, 0]
    s = sc_ref[...] + 0.0
    u = lax.bitcast_convert_type(s, jnp.uint32)
    key = jnp.where((u >> 31) == jnp.uint32(1), ~u, u | jnp.uint32(0x80000000))
    key = jnp.where(mprev_ref[...] > 0.5, key, jnp.uint32(0))

    def body(_, state):
        lo_, hi_ = state
        mid = lo_ + (hi_ - lo_) // jnp.uint32(2) + jnp.uint32(1)
        cnt = jnp.sum((key >= mid).astype(jnp.int32))
        take = cnt >= k
        return (jnp.where(take, mid, lo_), jnp.where(take, hi_, mid - 1))

    lo_, _ = lax.fori_loop(0, 32, body, (jnp.uint32(0), jnp.uint32(0xFFFFFFFF)))
    gt = key > lo_
    eq = key == lo_
    need = (k - jnp.sum(gt.astype(jnp.int32))).astype(jnp.float32)
    eqf = eq.astype(jnp.float32)
    ii = lax.broadcasted_iota(jnp.int32, (128, 128), 0)
    jj = lax.broadcasted_iota(jnp.int32, (128, 128), 1)
    lt128 = (ii <= jj).astype(jnp.float32)
    rowc = eqf @ lt128
    rsum = rowc[:, 127:128]
    ri = lax.broadcasted_iota(jnp.int32, (80, 80), 0)
    rj = lax.broadcasted_iota(jnp.int32, (80, 80), 1)
    sl80 = (rj < ri).astype(jnp.float32)
    off = sl80 @ rsum
    rank = rowc + off - 1.0
    sel = gt | (eq & (rank < need))
    mnew_ref[...] = sel.astype(jnp.float32)


def _pool(score2d, mprev2d, k):
    return _tc_call(
        _pool_body, (1,),
        [_full((80, 128)), _full((80, 128)), _full((1, 1))],
        _full((80, 128)), jax.ShapeDtypeStruct((80, 128), jnp.float32),
    )(score2d, mprev2d, k)


def _gate_body(z_ref, sc_ref, mnew_ref, kinv_ref, x_ref, r_ref):
    i = pl.program_id(0)
    gate = jax.nn.sigmoid(sc_ref[...]) * mnew_ref[...]
    xn = z_ref[...] * gate
    x_ref[...] = xn
    ps = jnp.sum(xn, axis=0, keepdims=True)
    part = jnp.concatenate([ps * kinv_ref[0, 0], ps], axis=-1)

    @pl.when(i == 0)
    def _():
        r_ref[...] = jnp.zeros_like(r_ref)

    r_ref[...] += part


def _gate(z, score, mnew, kinv):
    return _tc_call(
        _gate_body, (8,),
        [_rows((BR, 256)), _rows((BR, 1)), _rows((BR, 1)), _full((1, 1))],
        (_rows((BR, 256)), _full((1, 512))),
        (jax.ShapeDtypeStruct((NP, 256), jnp.float32),
         jax.ShapeDtypeStruct((1, 512), jnp.float32)),
    )(z, score, mnew, kinv)


def _post_body(r1, r2, r3, a0, c0, a1, c1, out_ref):
    r = r1[...] + r2[...] + r3[...]
    h = jnp.tanh(r @ a0[...] + c0[...])
    out_ref[...] = h @ a1[...] + c1[...]


def _post(r1, r2, r3, a0, c0, a1, c1):
    return _tc_call(
        _post_body, (1,),
        [_full((1, 512))] * 3 + [_full((512, 256)), _full((1, 256)),
                                 _full((256, 128)), _full((1, 128))],
        _full((1, 128)), jax.ShapeDtypeStruct((1, 128), jnp.float32),
    )(r1, r2, r3, a0, c0, a1, c1)


def _fold_bn(W, b, g, be):
    sc = (g * _SB).astype(jnp.float32)
    return W * sc[None, :], (b * sc + be)[None, :]


def kernel(x, edge_index, i, params):
    P = params
    src = edge_index[0]
    dst = edge_index[1]
    x_pad = jnp.pad(x, ((0, NP - N), (0, 0)))

    a0, c0 = _fold_bn(P['pre_W0'], P['pre_b0'], P['pre_g0'], P['pre_be0'])
    a1, c1 = _fold_bn(P['pre_W1'], P['pre_b1'], P['pre_g1'], P['pre_be1'])
    a2, c2 = _fold_bn(P['pre_W2'], P['pre_b2'], P['pre_g2'], P['pre_be2'])
    pa0, pc0 = _fold_bn(P['post_W0'], P['post_b0'], P['post_g0'], P['post_be0'])
    pa1, pc1 = _fold_bn(P['post_W1'], P['post_b1'], P['post_g1'], P['post_be1'])
    bc = {k: P['conv_b%d' % k][None, :] for k in range(1, 6)}
    p1 = (P['p1'] / jnp.linalg.norm(P['p1']))[:, None]
    p2 = (P['p2'] / jnp.linalg.norm(P['p2']))[:, None]

    K1 = (N + 1) // 2
    K2 = (K1 + 1) // 2
    k1c = jnp.full((1, 1), K1, jnp.int32)
    k2c = jnp.full((1, 1), K2, jnp.int32)
    k1inv = jnp.full((1, 1), 1.0 / K1, jnp.float32)
    k2inv = jnp.full((1, 1), 1.0 / K2, jnp.float32)

    m0 = (jnp.arange(NP, dtype=jnp.int32) < N).astype(jnp.float32)
    m0c = m0.reshape(NP, 1)

    srcp0, dstp0, dp0 = _edge_prep(src, dst, m0)
    dinv0, dgi0 = _deg_reduce(dp0[:, :NP].reshape(NC, 80, 128))
    dinv0c, dgi0c = dinv0.reshape(NP, 1), dgi0.reshape(NP, 1)
    h = _pre_mlp(x_pad, a0, c0, a1, c1, a2, c2)
    y1, yh1 = _conv_pre(h, P['conv_W1'], dinv0c)
    ms1 = _msgpass(yh1, srcp0, dstp0)
    y2, yh2 = _combine_pre(ms1, y1, dinv0c, dgi0c, bc[1], m0c, P['conv_W2'])
    ms2 = _msgpass(yh2, srcp0, dstp0)
    z2, sc1 = _combine_score(ms2, y2, dinv0c, dgi0c, bc[2], m0c, p1)
    m1_2d = _pool(sc1.reshape(80, 128), m0.reshape(80, 128), k1c)
    m1 = m1_2d.reshape(NP)
    m1c = m1.reshape(NP, 1)
    x1, r1 = _gate(z2, sc1, m1c, k1inv)

    srcp1, dstp1, dp1 = _edge_prep(src, dst, m1)
    dinv1, dgi1 = _deg_reduce(dp1[:, :NP].reshape(NC, 80, 128))
    dinv1c, dgi1c = dinv1.reshape(NP, 1), dgi1.reshape(NP, 1)
    y3, yh3 = _conv_pre(x1, P['conv_W3'], dinv1c)
    ms3 = _msgpass(yh3, srcp1, dstp1)
    y4, yh4 = _combine_pre(ms3, y3, dinv1c, dgi1c, bc[3], m1c, P['conv_W4'])
    ms4 = _msgpass(yh4, srcp1, dstp1)
    z4, sc2 = _combine_score(ms4, y4, dinv1c, dgi1c, bc[4], m1c, p2)
    m2_2d = _pool(sc2.reshape(80, 128), m1_2d, k2c)
    m2 = m2_2d.reshape(NP)
    m2c = m2.reshape(NP, 1)
    x2, r2 = _gate(z4, sc2, m2c, k2inv)

    srcp2, dstp2, dp2 = _edge_prep(src, dst, m2)
    dinv2, dgi2 = _deg_reduce(dp2[:, :NP].reshape(NC, 80, 128))
    dinv2c, dgi2c = dinv2.reshape(NP, 1), dgi2.reshape(NP, 1)
    y5, yh5 = _conv_pre(x2, P['conv_W5'], dinv2c)
    ms5 = _msgpass(yh5, srcp2, dstp2)
    r3 = _combine_read(ms5, y5, dinv2c, dgi2c, bc[5], m2c, k2inv)

    return _post(r1, r2, r3, pa0, pc0, pa1, pc1)
